# baseline (speedup 1.0000x reference)
import functools

import jax
import jax.numpy as jnp
from jax import lax
from jax.experimental import pallas as pl
from jax.experimental.pallas import tpu as pltpu
from jax.experimental.pallas import tpu_sc as plsc

VOCAB = 1000
TPAD = 32
NC = 2
NS = 16
NW = NC * NS


def _make_gather(b, t):
    b_per_w = b // NW
    assert b_per_w % 2 == 0
    mesh = plsc.VectorSubcoreMesh(core_axis_name="c", subcore_axis_name="s")

    @functools.partial(
        pl.kernel,
        out_type=jax.ShapeDtypeStruct((b, t, VOCAB), jnp.float32),
        mesh=mesh,
        scratch_types=[
            pltpu.VMEM((b_per_w * TPAD,), jnp.int32),
            pltpu.VMEM((2, t, VOCAB), jnp.float32),
            pltpu.SemaphoreType.DMA,
            pltpu.SemaphoreType.DMA,
        ],
        compiler_params=pltpu.CompilerParams(use_tc_tiling_on_sc=False),
    )
    def gather_kernel(table_hbm, idx_hbm, out_hbm, idx_v, rows_v, sem0, sem1):
        wid = lax.axis_index("s") * NC + lax.axis_index("c")
        base = wid * b_per_w
        sems = (sem0, sem1)
        pltpu.sync_copy(idx_hbm.at[pl.ds(base * TPAD, b_per_w * TPAD)], idx_v)

        def gather_dma(c, slot):
            return pltpu.make_async_copy(
                table_hbm.at[idx_v.at[pl.ds(c * TPAD, t)]],
                rows_v.at[slot],
                sems[slot],
            )

        def out_copy(c, slot):
            pltpu.sync_copy(rows_v.at[slot], out_hbm.at[base + c])

        gather_dma(0, 0).start()

        def body(c2, carry):
            c = 2 * c2
            gather_dma(c + 1, 1).start()
            gather_dma(c, 0).wait()
            out_copy(c, 0)
            gather_dma(c + 2, 0).start()
            gather_dma(c + 1, 1).wait()
            out_copy(c + 1, 1)
            return carry

        lax.fori_loop(0, b_per_w // 2 - 1, body, 0)
        c = b_per_w - 2
        gather_dma(c + 1, 1).start()
        gather_dma(c, 0).wait()
        out_copy(c, 0)
        gather_dma(c + 1, 1).wait()
        out_copy(c + 1, 1)

    return gather_kernel


_gather = _make_gather(4096, 20)


@jax.jit
def kernel(idx, token_embedding_table):
    b, t = idx.shape
    idx_p = jnp.pad(idx, ((0, 0), (0, TPAD - t))).reshape(b * TPAD)
    return _gather(token_embedding_table, idx_p)
